# baseline (speedup 1.0000x reference)
import functools

import jax
import jax.numpy as jnp
from jax import lax
from jax.experimental import pallas as pl
from jax.experimental.pallas import tpu as pltpu
from jax.experimental.pallas import tpu_sc as plsc

B = 4096
V = 200
K = 128
D = 64

BH = B // 128
BH_SC = 8
BH_TC = BH - BH_SC
S = 32
CPS = BH_SC * 8 * 8 // S
GSZ = 4

NUM_CORES = 2
NUM_SUBCORES = 16
NW = NUM_CORES * NUM_SUBCORES
CQ = K * CPS
CPW = CQ // NW
ITERS = CPW // GSZ
NBUF = 4


def _sc_gather_half(xc, cidx):
  mesh = plsc.VectorSubcoreMesh(core_axis_name="c", subcore_axis_name="s")

  @functools.partial(
      pl.kernel,
      mesh=mesh,
      out_type=jax.ShapeDtypeStruct((CQ, S, 128), jnp.float32),
      scratch_types=[
          pltpu.VMEM((ITERS, GSZ), jnp.int32),
          [pltpu.VMEM((GSZ, S, 128), jnp.float32) for _ in range(NBUF)],
          [pltpu.SemaphoreType.DMA for _ in range(NBUF)],
          [pltpu.SemaphoreType.DMA for _ in range(NBUF)],
      ],
  )
  def k(x_hbm, cidx_hbm, out_hbm, idx_v, bufs, gsems, ssems):
    cid = lax.axis_index("c")
    sid = lax.axis_index("s")
    wid = sid * NUM_CORES + cid
    obase = wid * CPW

    pltpu.sync_copy(cidx_hbm.at[pl.ds(wid * ITERS, ITERS)], idx_v)

    def gather(i, b):
      return pltpu.make_async_copy(x_hbm.at[idx_v.at[i]], bufs[b], gsems[b])

    def store(i, b):
      return pltpu.make_async_copy(
          bufs[b], out_hbm.at[pl.ds(obase + i * GSZ, GSZ)], ssems[b])

    for b in range(NBUF):
      gather(b, b).start()

    def body(cc, carry):
      for kk in range(NBUF):
        i = cc * NBUF + kk
        gather(i, kk).wait()
        store(i, kk).start()

        @pl.when((i >= 1) & (i + NBUF - 1 < ITERS))
        def _():
          pb = (kk + NBUF - 1) % NBUF
          store(i - 1, pb).wait()
          gather(i + NBUF - 1, pb).start()

      return carry

    lax.fori_loop(0, ITERS // NBUF, body, 0)

    for t in range(NBUF):
      i = ITERS - NBUF + t
      store(i, i % NBUF).wait()

  return k(xc, cidx)


def _mxu_transpose(a):
  eye = (lax.broadcasted_iota(jnp.int32, (K, K), 0)
         == lax.broadcasted_iota(jnp.int32, (K, K), 1)).astype(jnp.float32)
  r = lax.dot_general(a.reshape(K, 64 * 128), eye, (((0,), (0,)), ((), ())),
                      preferred_element_type=jnp.float32)
  t = jnp.transpose(r.reshape(64, 128, K), (1, 0, 2))
  return t.reshape(1, 128, 8, 8, K)


BG = 2


def _tc_direct(idx32, x5):
  nsteps = BH_TC // BG

  def body(idx_s, x_hbm, out_ref, st0, st1, sem0, sem1):
    g = pl.program_id(0)

    def fire(gg, buf, sem):
      bh = BH_SC + gg * BG
      for j in range(K):
        pltpu.make_async_copy(
            x_hbm.at[idx_s[j], :, pl.ds(bh, BG)], buf.at[j], sem).start()

    def drain(gg, buf, sem):
      bh = BH_SC + gg * BG
      for j in range(K):
        pltpu.make_async_copy(
            x_hbm.at[idx_s[j], :, pl.ds(bh, BG)], buf.at[j], sem).wait()

    def compute(buf):
      for t in range(BG):
        out_ref[pl.ds(t, 1)] = _mxu_transpose(buf[:, :, t])

    @pl.when(g == 0)
    def _():
      fire(0, st0, sem0)

    @pl.when(lax.rem(g, 2) == 0)
    def _():
      @pl.when(g + 1 < nsteps)
      def _():
        fire(g + 1, st1, sem1)
      drain(g, st0, sem0)
      compute(st0)

    @pl.when(lax.rem(g, 2) == 1)
    def _():
      @pl.when(g + 1 < nsteps)
      def _():
        fire(g + 1, st0, sem0)
      drain(g, st1, sem1)
      compute(st1)

  grid_spec = pltpu.PrefetchScalarGridSpec(
      num_scalar_prefetch=1,
      grid=(nsteps,),
      in_specs=[pl.BlockSpec(memory_space=pl.ANY)],
      out_specs=pl.BlockSpec(
          (BG, 128, 8, 8, K), lambda g, idx: (BH_SC // BG + g, 0, 0, 0, 0)),
      scratch_shapes=[
          pltpu.VMEM((K, 8, BG, 8, 128), jnp.float32),
          pltpu.VMEM((K, 8, BG, 8, 128), jnp.float32),
          pltpu.SemaphoreType.DMA,
          pltpu.SemaphoreType.DMA,
      ],
  )
  return pl.pallas_call(
      body,
      grid_spec=grid_spec,
      out_shape=jax.ShapeDtypeStruct((BH, 128, 8, 8, K), jnp.float32),
      compiler_params=pltpu.CompilerParams(
          dimension_semantics=("arbitrary",)),
  )(idx32, x5)


def _tc_transpose_sc_half(interm5, prev_out):
  def body(in_ref, _prev, out_ref):
    out_ref[...] = _mxu_transpose(in_ref[:, :, 0, :, :])

  return pl.pallas_call(
      body,
      grid=(BH_SC,),
      in_specs=[
          pl.BlockSpec((K, 8, 1, 8, 128), lambda g: (0, 0, g, 0, 0)),
          pl.BlockSpec(memory_space=pl.ANY),
      ],
      out_specs=pl.BlockSpec(
          (1, 128, 8, 8, K), lambda g: (g, 0, 0, 0, 0)),
      out_shape=jax.ShapeDtypeStruct((BH, 128, 8, 8, K), jnp.float32),
      input_output_aliases={1: 0},
      compiler_params=pltpu.CompilerParams(
          dimension_semantics=("arbitrary",)),
  )(interm5, prev_out)


def kernel(x, index):
  x5 = (x.transpose(1, 2, 0)
        .reshape(V, D // 8, 8, B // 128, 128)
        .transpose(0, 1, 3, 2, 4))
  xc = x5.reshape(V * 8 * BH * 8 // S, S, 128)

  idx32 = index.astype(jnp.int32)
  offs = (jnp.arange(8, dtype=jnp.int32)[:, None] * 8
          + jnp.arange(CPS // 8, dtype=jnp.int32)[None, :]).reshape(-1)
  cidx = (idx32[:, None] * 64 + offs[None, :]).reshape(-1, GSZ)

  interm = _sc_gather_half(xc, cidx)
  interm5 = interm.reshape(K, 8, BH_SC, 8, 128)

  out = _tc_direct(idx32, x5)
  out = _tc_transpose_sc_half(interm5, out)

  return (out.transpose(0, 1, 4, 2, 3)
          .reshape(B, K, D))


# speedup vs baseline: 1.0922x; 1.0922x over previous
import functools

import jax
import jax.numpy as jnp
from jax import lax
from jax.experimental import pallas as pl
from jax.experimental.pallas import tpu as pltpu
from jax.experimental.pallas import tpu_sc as plsc

B = 4096
V = 200
K = 128
D = 64

BH = B // 128
BH_SC = 4
BH_TC = BH - BH_SC
S = 32
CPS = BH_SC * 8 * 8 // S
GSZ = 4

NUM_CORES = 2
NUM_SUBCORES = 16
NW = NUM_CORES * NUM_SUBCORES
CQ = K * CPS
CPW = CQ // NW
ITERS = CPW // GSZ
NBUF = 4


def _sc_gather_half(xc, cidx):
  mesh = plsc.VectorSubcoreMesh(core_axis_name="c", subcore_axis_name="s")

  @functools.partial(
      pl.kernel,
      mesh=mesh,
      out_type=jax.ShapeDtypeStruct((CQ, S, 128), jnp.float32),
      scratch_types=[
          pltpu.VMEM((ITERS, GSZ), jnp.int32),
          [pltpu.VMEM((GSZ, S, 128), jnp.float32) for _ in range(NBUF)],
          [pltpu.SemaphoreType.DMA for _ in range(NBUF)],
          [pltpu.SemaphoreType.DMA for _ in range(NBUF)],
      ],
  )
  def k(x_hbm, cidx_hbm, out_hbm, idx_v, bufs, gsems, ssems):
    cid = lax.axis_index("c")
    sid = lax.axis_index("s")
    wid = sid * NUM_CORES + cid
    obase = wid * CPW

    pltpu.sync_copy(cidx_hbm.at[pl.ds(wid * ITERS, ITERS)], idx_v)

    def gather(i, b):
      return pltpu.make_async_copy(x_hbm.at[idx_v.at[i]], bufs[b], gsems[b])

    def store(i, b):
      return pltpu.make_async_copy(
          bufs[b], out_hbm.at[pl.ds(obase + i * GSZ, GSZ)], ssems[b])

    for b in range(NBUF):
      gather(b, b).start()

    def body(cc, carry):
      for kk in range(NBUF):
        i = cc * NBUF + kk
        gather(i, kk).wait()
        store(i, kk).start()

        @pl.when((i >= 1) & (i + NBUF - 1 < ITERS))
        def _():
          pb = (kk + NBUF - 1) % NBUF
          store(i - 1, pb).wait()
          gather(i + NBUF - 1, pb).start()

      return carry

    lax.fori_loop(0, ITERS // NBUF, body, 0)

    for t in range(NBUF):
      i = ITERS - NBUF + t
      store(i, i % NBUF).wait()

  return k(xc, cidx)


def _mxu_transpose(a):
  eye = (lax.broadcasted_iota(jnp.int32, (K, K), 0)
         == lax.broadcasted_iota(jnp.int32, (K, K), 1)).astype(jnp.float32)
  r = lax.dot_general(a.reshape(K, 64 * 128), eye, (((0,), (0,)), ((), ())),
                      preferred_element_type=jnp.float32)
  t = jnp.transpose(r.reshape(64, 128, K), (1, 0, 2))
  return t.reshape(1, 128, 8, 8, K)


BG = 2


def _tc_direct(idx32, x5):
  nsteps = BH_TC // BG

  def body(idx_s, x_hbm, out_ref, st0, st1, sem0, sem1):
    g = pl.program_id(0)

    def fire(gg, buf, sem):
      bh = BH_SC + gg * BG
      for j in range(K):
        pltpu.make_async_copy(
            x_hbm.at[idx_s[j], :, pl.ds(bh, BG)], buf.at[j], sem).start()

    def drain(gg, buf, sem):
      bh = BH_SC + gg * BG
      for j in range(K):
        pltpu.make_async_copy(
            x_hbm.at[idx_s[j], :, pl.ds(bh, BG)], buf.at[j], sem).wait()

    def compute(buf):
      for t in range(BG):
        out_ref[pl.ds(t, 1)] = _mxu_transpose(buf[:, :, t])

    @pl.when(g == 0)
    def _():
      fire(0, st0, sem0)

    @pl.when(lax.rem(g, 2) == 0)
    def _():
      @pl.when(g + 1 < nsteps)
      def _():
        fire(g + 1, st1, sem1)
      drain(g, st0, sem0)
      compute(st0)

    @pl.when(lax.rem(g, 2) == 1)
    def _():
      @pl.when(g + 1 < nsteps)
      def _():
        fire(g + 1, st0, sem0)
      drain(g, st1, sem1)
      compute(st1)

  grid_spec = pltpu.PrefetchScalarGridSpec(
      num_scalar_prefetch=1,
      grid=(nsteps,),
      in_specs=[pl.BlockSpec(memory_space=pl.ANY)],
      out_specs=pl.BlockSpec(
          (BG, 128, 8, 8, K), lambda g, idx: (BH_SC // BG + g, 0, 0, 0, 0)),
      scratch_shapes=[
          pltpu.VMEM((K, 8, BG, 8, 128), jnp.float32),
          pltpu.VMEM((K, 8, BG, 8, 128), jnp.float32),
          pltpu.SemaphoreType.DMA,
          pltpu.SemaphoreType.DMA,
      ],
  )
  return pl.pallas_call(
      body,
      grid_spec=grid_spec,
      out_shape=jax.ShapeDtypeStruct((BH, 128, 8, 8, K), jnp.float32),
      compiler_params=pltpu.CompilerParams(
          dimension_semantics=("arbitrary",)),
  )(idx32, x5)


def _tc_transpose_sc_half(interm5, prev_out):
  def body(in_ref, _prev, out_ref):
    out_ref[...] = _mxu_transpose(in_ref[:, :, 0, :, :])

  return pl.pallas_call(
      body,
      grid=(BH_SC,),
      in_specs=[
          pl.BlockSpec((K, 8, 1, 8, 128), lambda g: (0, 0, g, 0, 0)),
          pl.BlockSpec(memory_space=pl.ANY),
      ],
      out_specs=pl.BlockSpec(
          (1, 128, 8, 8, K), lambda g: (g, 0, 0, 0, 0)),
      out_shape=jax.ShapeDtypeStruct((BH, 128, 8, 8, K), jnp.float32),
      input_output_aliases={1: 0},
      compiler_params=pltpu.CompilerParams(
          dimension_semantics=("arbitrary",)),
  )(interm5, prev_out)


def kernel(x, index):
  x5 = (x.transpose(1, 2, 0)
        .reshape(V, D // 8, 8, B // 128, 128)
        .transpose(0, 1, 3, 2, 4))
  xc = x5.reshape(V * 8 * BH * 8 // S, S, 128)

  idx32 = index.astype(jnp.int32)
  offs = (jnp.arange(8, dtype=jnp.int32)[:, None] * 8
          + jnp.arange(CPS // 8, dtype=jnp.int32)[None, :]).reshape(-1)
  cidx = (idx32[:, None] * 64 + offs[None, :]).reshape(-1, GSZ)

  interm = _sc_gather_half(xc, cidx)
  interm5 = interm.reshape(K, 8, BH_SC, 8, 128)

  out = _tc_direct(idx32, x5)
  out = _tc_transpose_sc_half(interm5, out)

  return (out.transpose(0, 1, 4, 2, 3)
          .reshape(B, K, D))
